# baseline (speedup 1.0000x reference)
import functools

import jax
import jax.numpy as jnp
from jax import lax
from jax.experimental import pallas as pl
from jax.experimental.pallas import tpu as pltpu
from jax.experimental.pallas import tpu_sc as plsc

_SEG = (96, 104)


@functools.lru_cache(maxsize=None)
def _make_kernel(B, S, D, V):
    info = plsc.get_sparse_core_info()
    NC, NS, L = info.num_cores, info.num_subcores, info.num_lanes
    NW = NC * NS
    BPW = B // NW
    NB = 2
    segs = []
    o = 0
    for n in _SEG:
        segs.append((o, n))
        o += n
    assert o == S and all(n % 8 == 0 and s % 8 == 0 and n <= 128
                          for s, n in segs)
    assert B % NW == 0 and BPW % NB == 0 and D % L == 0

    mesh = plsc.VectorSubcoreMesh(core_axis_name="c", subcore_axis_name="s")

    @functools.partial(
        pl.kernel,
        mesh=mesh,
        out_type=jax.ShapeDtypeStruct((B * S, D), jnp.float32),
        scratch_types=[
            pltpu.VMEM((BPW, 2, S // 2), jnp.int32),
            pltpu.VMEM((S, D), jnp.float32),
            pltpu.VMEM((NB, S, D), jnp.float32),
        ] + [pltpu.SemaphoreType.DMA] * (2 * NB),
    )
    def k(x_hbm, tok_hbm, pos_hbm, out_hbm, idx_v, pos_v, rows_v, *sems):
        sin, sout = sems[:NB], sems[NB:]
        wid = lax.axis_index("s") * NC + lax.axis_index("c")
        b0 = wid * BPW
        pltpu.sync_copy(pos_hbm, pos_v)
        pltpu.sync_copy(x_hbm.at[pl.ds(b0, BPW)], idx_v)

        H = S // 2

        def gather_copy(i, buf, j):
            return pltpu.make_async_copy(
                tok_hbm.at[idx_v.at[i, j]],
                rows_v.at[buf, pl.ds(j * H, H)], sin[buf])

        def gather_start(i, buf):
            for j in range(2):
                gather_copy(i, buf, j).start()

        def gather_wait(i, buf):
            for j in range(2):
                gather_copy(i, buf, j).wait()

        def out_start(i, buf, seg):
            s0, n = segs[seg]
            pltpu.async_copy(rows_v.at[buf, pl.ds(s0, n)],
                             out_hbm.at[pl.ds((b0 + i) * S + s0, n)],
                             sout[buf])

        def out_wait(buf):
            for s0, n in segs:
                pltpu.make_async_copy(rows_v.at[buf, pl.ds(s0, n)],
                                      out_hbm.at[pl.ds(s0, n)],
                                      sout[buf]).wait()

        def add_pos(buf, seg):
            s0, n = segs[seg]

            def body(r4, carry):
                for dr in range(4):
                    r = s0 + 4 * r4 + dr
                    for v in range(D // L):
                        sl = pl.ds(v * L, L)
                        plsc.addupdate(rows_v.at[buf, r, sl], pos_v[r, sl])
                return carry

            lax.fori_loop(0, n // 4, body, 0)

        gather_start(0, 0)

        def outer(t, carry):
            for b in range(NB):
                i = NB * t + b
                nb1 = (b + 1) % NB

                @pl.when(i >= 1)
                def _():
                    out_wait(nb1)

                @pl.when(i + 1 < BPW)
                def _():
                    gather_start(i + 1, nb1)

                gather_wait(i, b)
                for seg in range(len(segs)):
                    add_pos(b, seg)
                    out_start(i, b, seg)
            return carry

        lax.fori_loop(0, BPW // NB, outer, 0)
        out_wait((BPW - 1) % NB)

    return k


def kernel(x, token_table, pos_table):
    B, S = x.shape
    V, D = token_table.shape
    x2 = x.astype(jnp.int32).reshape(B, 2, S // 2)
    out = _make_kernel(B, S, D, V)(x2, token_table, pos_table)
    return out.reshape(B, S, D)


# speedup vs baseline: 1.0232x; 1.0232x over previous
import functools

import jax
import jax.numpy as jnp
from jax import lax
from jax.experimental import pallas as pl
from jax.experimental.pallas import tpu as pltpu
from jax.experimental.pallas import tpu_sc as plsc

_SEG = (64, 64, 72)


@functools.lru_cache(maxsize=None)
def _make_kernel(B, S, D, V):
    info = plsc.get_sparse_core_info()
    NC, NS, L = info.num_cores, info.num_subcores, info.num_lanes
    NW = NC * NS
    BPW = B // NW
    NB = 2
    segs = []
    o = 0
    for n in _SEG:
        segs.append((o, n))
        o += n
    assert o == S and all(n % 8 == 0 and s % 8 == 0 and n <= 128
                          for s, n in segs)
    assert B % NW == 0 and BPW % NB == 0 and D % L == 0

    mesh = plsc.VectorSubcoreMesh(core_axis_name="c", subcore_axis_name="s")

    @functools.partial(
        pl.kernel,
        mesh=mesh,
        out_type=jax.ShapeDtypeStruct((B * S, D), jnp.float32),
        scratch_types=[
            pltpu.VMEM((BPW, 2, S // 2), jnp.int32),
            pltpu.VMEM((S, D), jnp.float32),
            pltpu.VMEM((NB, S, D), jnp.float32),
        ] + [pltpu.SemaphoreType.DMA] * (2 * NB),
    )
    def k(x_hbm, tok_hbm, pos_hbm, out_hbm, idx_v, pos_v, rows_v, *sems):
        sin, sout = sems[:NB], sems[NB:]
        wid = lax.axis_index("s") * NC + lax.axis_index("c")
        b0 = wid * BPW
        pltpu.sync_copy(pos_hbm, pos_v)
        pltpu.sync_copy(x_hbm.at[pl.ds(b0, BPW)], idx_v)

        H = S // 2

        def gather_copy(i, buf, j):
            return pltpu.make_async_copy(
                tok_hbm.at[idx_v.at[i, j]],
                rows_v.at[buf, pl.ds(j * H, H)], sin[buf])

        def gather_start(i, buf):
            for j in range(2):
                gather_copy(i, buf, j).start()

        def gather_wait(i, buf):
            for j in range(2):
                gather_copy(i, buf, j).wait()

        def out_start(i, buf, seg):
            s0, n = segs[seg]
            pltpu.async_copy(rows_v.at[buf, pl.ds(s0, n)],
                             out_hbm.at[pl.ds((b0 + i) * S + s0, n)],
                             sout[buf])

        def out_wait(buf):
            for s0, n in segs:
                pltpu.make_async_copy(rows_v.at[buf, pl.ds(s0, n)],
                                      out_hbm.at[pl.ds(s0, n)],
                                      sout[buf]).wait()

        def add_pos(buf, seg):
            s0, n = segs[seg]

            @plsc.parallel_loop(s0, s0 + n, step=1, unroll=4)
            def _(r):
                for v in range(D // L):
                    sl = pl.ds(v * L, L)
                    plsc.addupdate(rows_v.at[buf, r, sl], pos_v[r, sl])

        gather_start(0, 0)

        def outer(t, carry):
            for b in range(NB):
                i = NB * t + b
                nb1 = (b + 1) % NB

                @pl.when(i >= 1)
                def _():
                    out_wait(nb1)

                @pl.when(i + 1 < BPW)
                def _():
                    gather_start(i + 1, nb1)

                gather_wait(i, b)
                for seg in range(len(segs)):
                    add_pos(b, seg)
                    out_start(i, b, seg)
            return carry

        lax.fori_loop(0, BPW // NB, outer, 0)
        out_wait((BPW - 1) % NB)

    return k


def kernel(x, token_table, pos_table):
    B, S = x.shape
    V, D = token_table.shape
    x2 = x.astype(jnp.int32).reshape(B, 2, S // 2)
    out = _make_kernel(B, S, D, V)(x2, token_table, pos_table)
    return out.reshape(B, S, D)


# speedup vs baseline: 1.0297x; 1.0064x over previous
import functools

import jax
import jax.numpy as jnp
from jax import lax
from jax.experimental import pallas as pl
from jax.experimental.pallas import tpu as pltpu
from jax.experimental.pallas import tpu_sc as plsc

_SEG = (48, 48, 48, 56)


@functools.lru_cache(maxsize=None)
def _make_kernel(B, S, D, V):
    info = plsc.get_sparse_core_info()
    NC, NS, L = info.num_cores, info.num_subcores, info.num_lanes
    NW = NC * NS
    BPW = B // NW
    NB = 2
    segs = []
    o = 0
    for n in _SEG:
        segs.append((o, n))
        o += n
    assert o == S and all(n % 8 == 0 and s % 8 == 0 and n <= 128
                          for s, n in segs)
    assert B % NW == 0 and BPW % NB == 0 and D % L == 0

    mesh = plsc.VectorSubcoreMesh(core_axis_name="c", subcore_axis_name="s")

    @functools.partial(
        pl.kernel,
        mesh=mesh,
        out_type=jax.ShapeDtypeStruct((B * S, D), jnp.float32),
        scratch_types=[
            pltpu.VMEM((BPW, 2, S // 2), jnp.int32),
            pltpu.VMEM((S, D), jnp.float32),
            pltpu.VMEM((NB, S, D), jnp.float32),
        ] + [pltpu.SemaphoreType.DMA] * (2 * NB),
    )
    def k(x_hbm, tok_hbm, pos_hbm, out_hbm, idx_v, pos_v, rows_v, *sems):
        sin, sout = sems[:NB], sems[NB:]
        wid = lax.axis_index("s") * NC + lax.axis_index("c")
        b0 = wid * BPW
        pltpu.sync_copy(pos_hbm, pos_v)
        pltpu.sync_copy(x_hbm.at[pl.ds(b0, BPW)], idx_v)

        H = S // 2

        def gather_copy(i, buf, j):
            return pltpu.make_async_copy(
                tok_hbm.at[idx_v.at[i, j]],
                rows_v.at[buf, pl.ds(j * H, H)], sin[buf])

        def gather_start(i, buf):
            for j in range(2):
                gather_copy(i, buf, j).start()

        def gather_wait(i, buf):
            for j in range(2):
                gather_copy(i, buf, j).wait()

        def out_start(i, buf, seg):
            s0, n = segs[seg]
            pltpu.async_copy(rows_v.at[buf, pl.ds(s0, n)],
                             out_hbm.at[pl.ds((b0 + i) * S + s0, n)],
                             sout[buf])

        def out_wait(buf):
            for s0, n in segs:
                pltpu.make_async_copy(rows_v.at[buf, pl.ds(s0, n)],
                                      out_hbm.at[pl.ds(s0, n)],
                                      sout[buf]).wait()

        def add_pos(buf, seg):
            s0, n = segs[seg]

            @plsc.parallel_loop(s0, s0 + n, step=1, unroll=4)
            def _(r):
                for v in range(D // L):
                    sl = pl.ds(v * L, L)
                    plsc.addupdate(rows_v.at[buf, r, sl], pos_v[r, sl])

        gather_start(0, 0)

        def outer(t, carry):
            for b in range(NB):
                i = NB * t + b
                nb1 = (b + 1) % NB

                @pl.when(i >= 1)
                def _():
                    out_wait(nb1)

                @pl.when(i + 1 < BPW)
                def _():
                    gather_start(i + 1, nb1)

                gather_wait(i, b)
                for seg in range(len(segs)):
                    add_pos(b, seg)
                    out_start(i, b, seg)
            return carry

        lax.fori_loop(0, BPW // NB, outer, 0)
        out_wait((BPW - 1) % NB)

    return k


def kernel(x, token_table, pos_table):
    B, S = x.shape
    V, D = token_table.shape
    x2 = x.astype(jnp.int32).reshape(B, 2, S // 2)
    out = _make_kernel(B, S, D, V)(x2, token_table, pos_table)
    return out.reshape(B, S, D)
